# Initial kernel scaffold; baseline (speedup 1.0000x reference)
import jax
import jax.numpy as jnp
from jax.experimental import pallas as pl


def kernel(x, norm_weight, score_weight):
    raise NotImplementedError("write your pallas kernel here")


# speedup vs baseline: 1.3679x; 1.3679x over previous
import jax
import jax.numpy as jnp
from jax.experimental import pallas as pl
from jax.experimental.pallas import tpu as pltpu

_EPS = 1.1920929e-07
_HB = 8


def _axis_pool_kernel(x_ref, cw_ref, nw_ref, o_ref, *, d, w):
    X = x_ref[0]
    cw = cw_ref[...]
    c = jnp.sum(X * X, axis=0, keepdims=True)
    t = jnp.sum(X * cw, axis=0, keepdims=True)
    r = jax.lax.rsqrt(c * (1.0 / d) + _EPS)
    s = t * r
    cols = []
    for hh in range(_HB):
        sl = slice(hh * w, (hh + 1) * w)
        sseg = s[:, sl]
        m = jnp.max(sseg, axis=1, keepdims=True)
        e = jnp.exp(sseg - m)
        denom = jnp.sum(e, axis=1, keepdims=True)
        g = e * (r[:, sl] / denom)
        cols.append(jnp.sum(X[:, sl] * g, axis=1, keepdims=True))
    pooled = jnp.concatenate(cols, axis=1).T
    o_ref[0] = pooled * nw_ref[...]


def kernel(x, norm_weight, score_weight):
    b, d, h, w = x.shape
    g = h // _HB
    xr = x.reshape(b, d, h * w)
    cw = (norm_weight * score_weight).reshape(d, 1)
    nw = norm_weight.reshape(1, d)
    from functools import partial
    return pl.pallas_call(
        partial(_axis_pool_kernel, d=d, w=w),
        out_shape=jax.ShapeDtypeStruct((b, h, d), x.dtype),
        grid=(b, g),
        in_specs=[
            pl.BlockSpec((1, d, _HB * w), lambda i, j: (i, 0, j)),
            pl.BlockSpec((d, 1), lambda i, j: (0, 0)),
            pl.BlockSpec((1, d), lambda i, j: (0, 0)),
        ],
        out_specs=pl.BlockSpec((1, _HB, d), lambda i, j: (i, j, 0)),
        compiler_params=pltpu.CompilerParams(
            dimension_semantics=("parallel", "arbitrary"),
            vmem_limit_bytes=48 * 1024 * 1024,
        ),
        name="axis_attn_pool",
    )(xr, cw, nw)


# speedup vs baseline: 4.5730x; 3.3430x over previous
import jax
import jax.numpy as jnp
from jax.experimental import pallas as pl
from jax.experimental.pallas import tpu as pltpu

_EPS = 1.1920929e-07
_HB = 8


def _axis_pool_kernel(x_ref, cw_ref, nw_ref, o_ref, *, d):
    X = x_ref[0]
    cw = cw_ref[...]
    c = jnp.sum(X * X, axis=0)
    t = jnp.sum(X * cw, axis=0)
    r = jax.lax.rsqrt(c * (1.0 / d) + _EPS)
    s = t * r
    m = jnp.max(s, axis=1, keepdims=True)
    e = jnp.exp(s - m)
    denom = jnp.sum(e, axis=1, keepdims=True)
    g = e * (r / denom)
    pooled = jnp.sum(X * g[None], axis=2)
    o_ref[0] = pooled.T * nw_ref[...]


def kernel(x, norm_weight, score_weight):
    b, d, h, w = x.shape
    g = h // _HB
    cw = (norm_weight * score_weight).reshape(d, 1, 1)
    nw = norm_weight.reshape(1, d)
    from functools import partial
    return pl.pallas_call(
        partial(_axis_pool_kernel, d=d),
        out_shape=jax.ShapeDtypeStruct((b, h, d), x.dtype),
        grid=(b, g),
        in_specs=[
            pl.BlockSpec((1, d, _HB, w), lambda i, j: (i, 0, j, 0)),
            pl.BlockSpec((d, 1, 1), lambda i, j: (0, 0, 0)),
            pl.BlockSpec((1, d), lambda i, j: (0, 0)),
        ],
        out_specs=pl.BlockSpec((1, _HB, d), lambda i, j: (i, j, 0)),
        compiler_params=pltpu.CompilerParams(
            dimension_semantics=("parallel", "arbitrary"),
            vmem_limit_bytes=48 * 1024 * 1024,
        ),
        name="axis_attn_pool",
    )(x, cw, nw)


# speedup vs baseline: 4.5914x; 1.0040x over previous
import jax
import jax.numpy as jnp
from jax.experimental import pallas as pl
from jax.experimental.pallas import tpu as pltpu
from functools import partial

_EPS = 1.1920929e-07
_HB = 8


def _axis_pool_kernel(x_hbm, cw_ref, nw_ref, o_ref, buf, sem, *, d, nb, ng):
    nsteps = nb * ng
    cw = cw_ref[...]
    nw = nw_ref[...]

    def copy(i, slot):
        b = i // ng
        g = jax.lax.rem(i, ng)
        return pltpu.make_async_copy(
            x_hbm.at[b, :, pl.ds(g * _HB, _HB), :], buf.at[slot], sem.at[slot]
        )

    copy(0, 0).start()

    def body(i, carry):
        cur = jax.lax.rem(i, 2)
        nxt = jax.lax.rem(i + 1, 2)

        @pl.when(i + 1 < nsteps)
        def _():
            copy(i + 1, nxt).start()

        copy(i, cur).wait()
        X = buf[cur]
        c = jnp.sum(X * X, axis=0)
        t = jnp.sum(X * cw, axis=0)
        r = jax.lax.rsqrt(c * (1.0 / d) + _EPS)
        s = t * r
        m = jnp.max(s, axis=1, keepdims=True)
        e = jnp.exp(s - m)
        denom = jnp.sum(e, axis=1, keepdims=True)
        g2 = e * (r / denom)
        pooled = jnp.sum(X * g2[None], axis=2)
        b = i // ng
        g = jax.lax.rem(i, ng)
        o_ref[b, pl.ds(g * _HB, _HB), :] = pooled.T * nw
        return 0

    jax.lax.fori_loop(0, nsteps, body, 0)


def kernel(x, norm_weight, score_weight):
    b, d, h, w = x.shape
    ng = h // _HB
    cw = (norm_weight * score_weight).reshape(d, 1, 1)
    nw = norm_weight.reshape(1, d)
    return pl.pallas_call(
        partial(_axis_pool_kernel, d=d, nb=b, ng=ng),
        out_shape=jax.ShapeDtypeStruct((b, h, d), x.dtype),
        in_specs=[
            pl.BlockSpec(memory_space=pl.ANY),
            pl.BlockSpec(memory_space=pltpu.VMEM),
            pl.BlockSpec(memory_space=pltpu.VMEM),
        ],
        out_specs=pl.BlockSpec(memory_space=pltpu.VMEM),
        scratch_shapes=[
            pltpu.VMEM((2, d, _HB, w), jnp.float32),
            pltpu.SemaphoreType.DMA((2,)),
        ],
        compiler_params=pltpu.CompilerParams(
            vmem_limit_bytes=48 * 1024 * 1024,
        ),
        name="axis_attn_pool",
    )(x, cw, nw)
